# baseline (speedup 1.0000x reference)
import functools

import jax
import jax.numpy as jnp
from jax import lax
from jax.experimental import pallas as pl
from jax.experimental.pallas import tpu as pltpu
from jax.experimental.pallas import tpu_sc as plsc

N_CORES = 2
N_SUBCORES = 16
NW = N_CORES * N_SUBCORES
L = 16

BATCH = 16384
D = 64
BPW = BATCH // NW
CHUNK = 128
N_CHUNKS = BPW // CHUNK
GROUPS = BPW // L


def _permute(v, idx):
    dn = lax.GatherDimensionNumbers(offset_dims=(), collapsed_slice_dims=(0,),
                                    start_index_map=(0,))
    return lax.gather(v, idx[:, None], dn, (1,),
                      mode=lax.GatherScatterMode.PROMISE_IN_BOUNDS)


def _sc_body(nid_hbm, nnid_hbm, table_hbm, bias_hbm, out_hbm,
             idx_a, idx_b, rows_a, rows_b, bias_v, out_v, sem):
    wid = lax.axis_index("s") * N_CORES + lax.axis_index("c")
    base = wid * BPW

    pltpu.sync_copy(nid_hbm.at[wid], idx_a)
    pltpu.sync_copy(nnid_hbm.at[wid], idx_b)

    copies = []
    for c in range(N_CHUNKS):
        rows = pl.ds(c * CHUNK, CHUNK)
        copies.append(pltpu.async_copy(table_hbm.at[idx_a.at[c]],
                                       rows_a.at[rows], sem))
        copies.append(pltpu.async_copy(table_hbm.at[idx_b.at[c]],
                                       rows_b.at[rows], sem))
        copies.append(pltpu.async_copy(bias_hbm.at[idx_b.at[c]],
                                       bias_v.at[rows], sem))
    for cp in copies:
        cp.wait()

    lanes = lax.iota(jnp.int32, L)

    def group_body(g, carry):
        rbase = g * L
        vs = []
        for r in range(L):
            row = rbase + r
            acc = rows_a[row, pl.ds(0, L)] * rows_b[row, pl.ds(0, L)]
            for k in range(1, D // L):
                acc = acc + (rows_a[row, pl.ds(k * L, L)] *
                             rows_b[row, pl.ds(k * L, L)])
            vs.append(acc)
        s = L // 2
        while s >= 1:
            mask = (lanes & s) == 0
            pidx = lanes ^ s
            nxt = []
            for i in range(s):
                a = vs[i] + _permute(vs[i], pidx)
                b = vs[i + s] + _permute(vs[i + s], pidx)
                nxt.append(jnp.where(mask, a, b))
            vs = nxt
            s //= 2
        out_v[pl.ds(rbase, L)] = vs[0] + bias_v[pl.ds(rbase, L)]
        return carry

    lax.fori_loop(0, GROUPS, group_body, 0)

    pltpu.sync_copy(out_v, out_hbm.at[pl.ds(base, BPW)])


@jax.jit
def _sc_call(nid, nnid, table, bias):
    mesh = plsc.VectorSubcoreMesh(core_axis_name="c", subcore_axis_name="s")
    return pl.kernel(
        _sc_body,
        out_type=jax.ShapeDtypeStruct((BATCH,), jnp.float32),
        mesh=mesh,
        scratch_types=[
            pltpu.VMEM((N_CHUNKS, CHUNK), jnp.int32),
            pltpu.VMEM((N_CHUNKS, CHUNK), jnp.int32),
            pltpu.VMEM((BPW, D), jnp.float32),
            pltpu.VMEM((BPW, D), jnp.float32),
            pltpu.VMEM((BPW,), jnp.float32),
            pltpu.VMEM((BPW,), jnp.float32),
            pltpu.SemaphoreType.DMA,
        ],
        compiler_params=pltpu.CompilerParams(use_tc_tiling_on_sc=False),
    )(nid, nnid, table, bias)


def kernel(node_id, node_neighbor_id, embedding_matrix, bias):
    nid = node_id.astype(jnp.int32).reshape(NW, N_CHUNKS, CHUNK)
    nnid = node_neighbor_id.astype(jnp.int32).reshape(NW, N_CHUNKS, CHUNK)
    return _sc_call(nid, nnid, embedding_matrix, bias)
